# baseline (device time: 108921 ns/iter reference)
import jax
import jax.numpy as jnp
from jax import lax
from jax.experimental import pallas as pl
from jax.experimental.pallas import tpu as pltpu

T = 1024
D = 2048
V_LOCAL = 16384
CHUNK = 2048
NC = V_LOCAL // CHUNK


def kernel(x, W, labels):
    labels2d = labels.reshape(T, 1)

    def body(x_ref, w_ref, lbl_ref, out_ref,
             xb_ref, s_acc, l_acc, comm_send, comm_recv, send_sem, recv_sem):
        c = pl.program_id(0)
        my_x = lax.axis_index("x")
        my_y = lax.axis_index("y")
        my_z = lax.axis_index("z")
        partner = (my_x, 1 - my_y, my_z)

        @pl.when(c == 0)
        def _init():
            xb_ref[...] = x_ref[...].astype(jnp.bfloat16)
            s_acc[...] = jnp.zeros_like(s_acc)
            l_acc[...] = jnp.zeros_like(l_acc)

        wb = w_ref[...].astype(jnp.bfloat16)
        logits = jnp.dot(xb_ref[...], wb, preferred_element_type=jnp.float32)

        lb = logits.astype(jnp.bfloat16)
        eb = jnp.exp(lb)
        ones_col = jnp.ones((CHUNK, 1), jnp.bfloat16)
        s_acc[...] += jnp.dot(eb, ones_col, preferred_element_type=jnp.float32)

        col0 = my_y * V_LOCAL + c * CHUNK
        cols = col0 + lax.broadcasted_iota(jnp.int32, (T, CHUNK), 1)
        hit = cols == lbl_ref[...]
        masked = jnp.where(hit, lb, jnp.bfloat16(0.0))
        l_acc[...] += jnp.dot(masked, ones_col, preferred_element_type=jnp.float32)

        @pl.when(c == NC - 1)
        def _finish():
            comm_send[0:1, :] = s_acc[...].reshape(1, T)
            comm_send[1:2, :] = l_acc[...].reshape(1, T)

            barrier = pltpu.get_barrier_semaphore()
            pl.semaphore_signal(barrier, inc=1, device_id=partner,
                                device_id_type=pl.DeviceIdType.MESH)
            pl.semaphore_wait(barrier, 1)

            rdma = pltpu.make_async_remote_copy(
                src_ref=comm_send,
                dst_ref=comm_recv,
                send_sem=send_sem,
                recv_sem=recv_sem,
                device_id=partner,
                device_id_type=pl.DeviceIdType.MESH,
            )
            rdma.start()
            rdma.wait()

            s_tot = comm_send[0:1, :] + comm_recv[0:1, :]
            l_tot = comm_send[1:2, :] + comm_recv[1:2, :]
            out_ref[...] = jnp.log(s_tot) - l_tot

    out = pl.pallas_call(
        body,
        grid=(NC,),
        out_shape=jax.ShapeDtypeStruct((1, T), jnp.float32),
        in_specs=[
            pl.BlockSpec((T, D), lambda i: (0, 0)),
            pl.BlockSpec((D, CHUNK), lambda i: (0, i)),
            pl.BlockSpec((T, 1), lambda i: (0, 0)),
        ],
        out_specs=pl.BlockSpec((1, T), lambda i: (0, 0)),
        scratch_shapes=[
            pltpu.VMEM((T, D), jnp.bfloat16),
            pltpu.VMEM((T, 1), jnp.float32),
            pltpu.VMEM((T, 1), jnp.float32),
            pltpu.VMEM((2, T), jnp.float32),
            pltpu.VMEM((2, T), jnp.float32),
            pltpu.SemaphoreType.DMA,
            pltpu.SemaphoreType.DMA,
        ],
        compiler_params=pltpu.CompilerParams(
            dimension_semantics=("arbitrary",),
            collective_id=0,
            vmem_limit_bytes=64 * 1024 * 1024,
        ),
    )(x, W, labels2d)
    return out.reshape(T)


# device time: 63886 ns/iter; 1.7049x vs baseline; 1.7049x over previous
import jax
import jax.numpy as jnp
from jax import lax
from jax.experimental import pallas as pl
from jax.experimental.pallas import tpu as pltpu

T = 1024
D = 2048
V_LOCAL = 16384
CHUNK = 2048
NC = V_LOCAL // CHUNK


def kernel(x, W, labels):
    labels2d = labels.reshape(T, 1)

    def body(x_ref, w_ref, lbl_ref, out_ref,
             xb_ref, s_acc, l_acc, comm_send, comm_recv, send_sem, recv_sem):
        c = pl.program_id(0)
        my_x = lax.axis_index("x")
        my_y = lax.axis_index("y")
        my_z = lax.axis_index("z")
        partner = (my_x, 1 - my_y, my_z)

        @pl.when(c == 0)
        def _init():
            xb_ref[...] = (x_ref[...] * 16.0).astype(jnp.float8_e4m3fn)
            s_acc[...] = jnp.zeros_like(s_acc)
            l_acc[...] = jnp.zeros_like(l_acc)

        w8 = w_ref[...].astype(jnp.float8_e4m3fn)
        logits = jnp.dot(xb_ref[...], w8, preferred_element_type=jnp.float32)

        s_acc[...] += jnp.sum(jnp.exp(logits * (1.0 / 16.0)),
                              axis=1, keepdims=True)

        col0 = my_y * V_LOCAL + c * CHUNK
        cols = col0 + lax.broadcasted_iota(jnp.int32, (T, CHUNK), 1)
        hit = cols == lbl_ref[...]
        l_acc[...] += jnp.sum(jnp.where(hit, logits, 0.0), axis=1, keepdims=True)

        @pl.when(c == NC - 1)
        def _finish():
            comm_send[0:1, :] = s_acc[...].reshape(1, T)
            comm_send[1:2, :] = (l_acc[...] * (1.0 / 16.0)).reshape(1, T)

            barrier = pltpu.get_barrier_semaphore()
            pl.semaphore_signal(barrier, inc=1, device_id=partner,
                                device_id_type=pl.DeviceIdType.MESH)
            pl.semaphore_wait(barrier, 1)

            rdma = pltpu.make_async_remote_copy(
                src_ref=comm_send,
                dst_ref=comm_recv,
                send_sem=send_sem,
                recv_sem=recv_sem,
                device_id=partner,
                device_id_type=pl.DeviceIdType.MESH,
            )
            rdma.start()
            rdma.wait()

            s_tot = comm_send[0:1, :] + comm_recv[0:1, :]
            l_tot = comm_send[1:2, :] + comm_recv[1:2, :]
            out_ref[...] = jnp.log(s_tot) - l_tot

    out = pl.pallas_call(
        body,
        grid=(NC,),
        out_shape=jax.ShapeDtypeStruct((1, T), jnp.float32),
        in_specs=[
            pl.BlockSpec((T, D), lambda i: (0, 0)),
            pl.BlockSpec((D, CHUNK), lambda i: (0, i)),
            pl.BlockSpec((T, 1), lambda i: (0, 0)),
        ],
        out_specs=pl.BlockSpec((1, T), lambda i: (0, 0)),
        scratch_shapes=[
            pltpu.VMEM((T, D), jnp.float8_e4m3fn),
            pltpu.VMEM((T, 1), jnp.float32),
            pltpu.VMEM((T, 1), jnp.float32),
            pltpu.VMEM((2, T), jnp.float32),
            pltpu.VMEM((2, T), jnp.float32),
            pltpu.SemaphoreType.DMA,
            pltpu.SemaphoreType.DMA,
        ],
        compiler_params=pltpu.CompilerParams(
            dimension_semantics=("arbitrary",),
            collective_id=0,
            vmem_limit_bytes=64 * 1024 * 1024,
        ),
    )(x, W, labels2d)
    return out.reshape(T)


# device time: 63813 ns/iter; 1.7069x vs baseline; 1.0011x over previous
import jax
import jax.numpy as jnp
from jax import lax
from jax.experimental import pallas as pl
from jax.experimental.pallas import tpu as pltpu

T = 1024
D = 2048
V_LOCAL = 16384
CHUNK = 2048
NC = V_LOCAL // CHUNK


def kernel(x, W, labels):
    labels2d = labels.reshape(T, 1)

    def body(x_ref, w_ref, lbl_ref, out_ref,
             xb_ref, s_acc, l_acc, comm_send, comm_recv, send_sem, recv_sem):
        c = pl.program_id(0)
        my_x = lax.axis_index("x")
        my_y = lax.axis_index("y")
        my_z = lax.axis_index("z")
        partner = (my_x, 1 - my_y, my_z)

        @pl.when(c == 0)
        def _init():
            xb_ref[...] = x_ref[...].astype(jnp.float8_e4m3fn)
            s_acc[...] = jnp.zeros_like(s_acc)
            l_acc[...] = jnp.zeros_like(l_acc)

        w8 = w_ref[...].astype(jnp.float8_e4m3fn)
        logits = jnp.dot(xb_ref[...], w8, preferred_element_type=jnp.float32)

        s_acc[...] += jnp.sum(jnp.exp(logits), axis=1, keepdims=True)

        col0 = my_y * V_LOCAL + c * CHUNK
        cols = col0 + lax.broadcasted_iota(jnp.int32, (T, CHUNK), 1)
        hit = cols == lbl_ref[...]
        l_acc[...] += jnp.sum(jnp.where(hit, logits, 0.0), axis=1, keepdims=True)

        @pl.when(c == NC - 1)
        def _finish():
            comm_send[0:1, :] = s_acc[...].reshape(1, T)
            comm_send[1:2, :] = l_acc[...].reshape(1, T)

            barrier = pltpu.get_barrier_semaphore()
            pl.semaphore_signal(barrier, inc=1, device_id=partner,
                                device_id_type=pl.DeviceIdType.MESH)
            pl.semaphore_wait(barrier, 1)

            rdma = pltpu.make_async_remote_copy(
                src_ref=comm_send,
                dst_ref=comm_recv,
                send_sem=send_sem,
                recv_sem=recv_sem,
                device_id=partner,
                device_id_type=pl.DeviceIdType.MESH,
            )
            rdma.start()
            rdma.wait()

            s_tot = comm_send[0:1, :] + comm_recv[0:1, :]
            l_tot = comm_send[1:2, :] + comm_recv[1:2, :]
            out_ref[...] = jnp.log(s_tot) - l_tot

    out = pl.pallas_call(
        body,
        grid=(NC,),
        out_shape=jax.ShapeDtypeStruct((1, T), jnp.float32),
        in_specs=[
            pl.BlockSpec((T, D), lambda i: (0, 0)),
            pl.BlockSpec((D, CHUNK), lambda i: (0, i)),
            pl.BlockSpec((T, 1), lambda i: (0, 0)),
        ],
        out_specs=pl.BlockSpec((1, T), lambda i: (0, 0)),
        scratch_shapes=[
            pltpu.VMEM((T, D), jnp.float8_e4m3fn),
            pltpu.VMEM((T, 1), jnp.float32),
            pltpu.VMEM((T, 1), jnp.float32),
            pltpu.VMEM((2, T), jnp.float32),
            pltpu.VMEM((2, T), jnp.float32),
            pltpu.SemaphoreType.DMA,
            pltpu.SemaphoreType.DMA,
        ],
        compiler_params=pltpu.CompilerParams(
            dimension_semantics=("arbitrary",),
            collective_id=0,
            vmem_limit_bytes=64 * 1024 * 1024,
        ),
    )(x, W, labels2d)
    return out.reshape(T)


# device time: 63400 ns/iter; 1.7180x vs baseline; 1.0065x over previous
import jax
import jax.numpy as jnp
from jax import lax
from jax.experimental import pallas as pl
from jax.experimental.pallas import tpu as pltpu

T = 1024
D = 2048
V_LOCAL = 16384
CHUNK = 2048
NC = V_LOCAL // CHUNK


def kernel(x, W, labels):
    labels2d = labels.reshape(T, 1)

    def body(x_ref, w_ref, lbl_ref, out_ref,
             xb_ref, s_acc, l_acc, comm_send, comm_recv, send_sem, recv_sem):
        c = pl.program_id(0)
        my_x = lax.axis_index("x")
        my_y = lax.axis_index("y")
        my_z = lax.axis_index("z")
        partner = (my_x, 1 - my_y, my_z)

        @pl.when(c == 0)
        def _init():
            barrier = pltpu.get_barrier_semaphore()
            pl.semaphore_signal(barrier, inc=1, device_id=partner,
                                device_id_type=pl.DeviceIdType.MESH)
            pl.semaphore_wait(barrier, 1)
            xb_ref[...] = x_ref[...].astype(jnp.float8_e4m3fn)
            s_acc[...] = jnp.zeros_like(s_acc)
            l_acc[...] = jnp.zeros_like(l_acc)

        w8 = w_ref[...].astype(jnp.float8_e4m3fn)
        logits = jnp.dot(xb_ref[...], w8, preferred_element_type=jnp.float32)

        s_acc[...] += jnp.sum(jnp.exp(logits), axis=1, keepdims=True)

        col0 = my_y * V_LOCAL + c * CHUNK
        cols = col0 + lax.broadcasted_iota(jnp.int32, (T, CHUNK), 1)
        hit = cols == lbl_ref[...]
        l_acc[...] += jnp.sum(jnp.where(hit, logits, 0.0), axis=1, keepdims=True)

        @pl.when(c == NC - 1)
        def _finish():
            comm_send[0:1, :] = s_acc[...].reshape(1, T)
            comm_send[1:2, :] = l_acc[...].reshape(1, T)

            rdma = pltpu.make_async_remote_copy(
                src_ref=comm_send,
                dst_ref=comm_recv,
                send_sem=send_sem,
                recv_sem=recv_sem,
                device_id=partner,
                device_id_type=pl.DeviceIdType.MESH,
            )
            rdma.start()
            rdma.wait()

            s_tot = comm_send[0:1, :] + comm_recv[0:1, :]
            l_tot = comm_send[1:2, :] + comm_recv[1:2, :]
            out_ref[...] = jnp.log(s_tot) - l_tot

    out = pl.pallas_call(
        body,
        grid=(NC,),
        out_shape=jax.ShapeDtypeStruct((1, T), jnp.float32),
        in_specs=[
            pl.BlockSpec((T, D), lambda i: (0, 0)),
            pl.BlockSpec((D, CHUNK), lambda i: (0, i)),
            pl.BlockSpec((T, 1), lambda i: (0, 0)),
        ],
        out_specs=pl.BlockSpec((1, T), lambda i: (0, 0)),
        scratch_shapes=[
            pltpu.VMEM((T, D), jnp.float8_e4m3fn),
            pltpu.VMEM((T, 1), jnp.float32),
            pltpu.VMEM((T, 1), jnp.float32),
            pltpu.VMEM((2, T), jnp.float32),
            pltpu.VMEM((2, T), jnp.float32),
            pltpu.SemaphoreType.DMA,
            pltpu.SemaphoreType.DMA,
        ],
        compiler_params=pltpu.CompilerParams(
            dimension_semantics=("arbitrary",),
            collective_id=0,
            vmem_limit_bytes=64 * 1024 * 1024,
        ),
    )(x, W, labels2d)
    return out.reshape(T)
